# baseline (speedup 1.0000x reference)
import functools
import math

import jax
import jax.numpy as jnp
from jax import lax
from jax.experimental import pallas as pl
from jax.experimental.pallas import tpu as pltpu

_B, _T, _C = 1, 2048, 1024
_H, _HD = 16, 64
_E, _TOPK, _FF = 8, 2, 4096
_HALF = _HD // 2
_SQRT2 = math.sqrt(2.0)


def _layernorm(x, w, b, eps=1e-5, clip=65000.0):
    mu = jnp.mean(x, axis=-1, keepdims=True)
    xc = x - mu
    var = jnp.mean(xc * xc, axis=-1, keepdims=True)
    y = xc * lax.rsqrt(var + eps)
    y = jnp.clip(y, -clip, clip)
    return y * w + b


def _gelu_exact(x):
    return 0.5 * x * (1.0 + lax.erf(x / _SQRT2))


def _ln_body(x_ref, w_ref, b_ref, o_ref):
    o_ref[...] = _layernorm(x_ref[...], w_ref[...], b_ref[...])


def _ln_call(x, w, b):
    return pl.pallas_call(
        _ln_body,
        out_shape=jax.ShapeDtypeStruct((_T, _C), jnp.float32),
    )(x, w.reshape(1, _C), b.reshape(1, _C))


def _qkv_body(ln_ref, w_ref, sin_ref, cos_ref, q_ref, k_ref, v_ref):
    x = ln_ref[...]
    dn = (((1,), (1,)), ((), ()))
    y = lax.dot_general(x, w_ref[0], dn,
                        preferred_element_type=jnp.float32)
    sin = sin_ref[...]
    cos = cos_ref[...]

    def rope(t):
        t1 = t[:, :_HALF]
        t2 = t[:, _HALF:]
        return jnp.concatenate([t1 * cos - t2 * sin, t1 * sin + t2 * cos],
                               axis=-1)

    q_ref[0] = rope(y[:, :_HD])
    k_ref[0] = rope(y[:, _HD:2 * _HD])
    v_ref[0] = y[:, 2 * _HD:]


def _qkv_call(ln1, qkv_w, sin, cos):
    return pl.pallas_call(
        _qkv_body,
        grid=(_H,),
        in_specs=[
            pl.BlockSpec((_T, _C), lambda h: (0, 0)),
            pl.BlockSpec((1, 3 * _HD, _C), lambda h: (h, 0, 0)),
            pl.BlockSpec((_T, _HALF), lambda h: (0, 0)),
            pl.BlockSpec((_T, _HALF), lambda h: (0, 0)),
        ],
        out_specs=[
            pl.BlockSpec((1, _T, _HD), lambda h: (h, 0, 0)),
            pl.BlockSpec((1, _T, _HD), lambda h: (h, 0, 0)),
            pl.BlockSpec((1, _T, _HD), lambda h: (h, 0, 0)),
        ],
        out_shape=[jax.ShapeDtypeStruct((_H, _T, _HD), jnp.float32)] * 3,
    )(ln1, qkv_w.reshape(_H, 3 * _HD, _C), sin, cos)


_BQ = 512
_BK = 256


def _attn_body(q_ref, k_ref, v_ref, o_ref):
    qb = pl.program_id(1)
    q = q_ref[0]
    scale = 1.0 / math.sqrt(_HD)
    dn = (((1,), (1,)), ((), ()))

    def process(klen):
        s = lax.dot_general(q, k_ref[0, :klen, :], dn,
                            preferred_element_type=jnp.float32)
        s = s * scale
        col = lax.broadcasted_iota(jnp.int32, (_BQ, klen), 1)
        row = qb * _BQ + lax.broadcasted_iota(jnp.int32, (_BQ, klen), 0)
        s = jnp.where(col <= row, s, -1e30)
        m = jnp.max(s, axis=-1, keepdims=True)
        p = jnp.exp(s - m)
        l = jnp.sum(p, axis=-1, keepdims=True)
        ctx = lax.dot_general(
            p, v_ref[0, :klen, :], (((1,), (0,)), ((), ())),
            preferred_element_type=jnp.float32)
        o_ref[0] = ctx / l

    nspan = 1
    for i in range(_T // _BQ // nspan):
        @pl.when(qb // nspan == i)
        def _():
            process((i + 1) * nspan * _BQ)


def _attn_call(q, k, v):
    return pl.pallas_call(
        _attn_body,
        grid=(_H, _T // _BQ),
        in_specs=[
            pl.BlockSpec((1, _BQ, _HD), lambda h, qb: (h, qb, 0)),
            pl.BlockSpec((1, _T, _HD), lambda h, qb: (h, 0, 0)),
            pl.BlockSpec((1, _T, _HD), lambda h, qb: (h, 0, 0)),
        ],
        out_specs=pl.BlockSpec((1, _BQ, _HD), lambda h, qb: (h, qb, 0)),
        out_shape=jax.ShapeDtypeStruct((_H, _T, _HD), jnp.float32),
    )(q, k, v)


def _proj_body(ctx_ref, pw_ref, x_ref, w2_ref, b2_ref, gw_ref,
               x1_ref, h2_ref, ind_ref, val_ref, acc_ref):
    h = pl.program_id(0)

    @pl.when(h == 0)
    def _():
        acc_ref[...] = jnp.zeros_like(acc_ref)

    dn = (((1,), (1,)), ((), ()))
    acc_ref[...] += lax.dot_general(ctx_ref[0], pw_ref[0], dn,
                                    preferred_element_type=jnp.float32)

    @pl.when(h == _H - 1)
    def _():
        x1 = x_ref[...] + acc_ref[...]
        x1_ref[...] = x1
        h2 = _layernorm(x1, w2_ref[...], b2_ref[...])
        h2_ref[...] = h2
        logits = lax.dot_general(h2, gw_ref[...], dn,
                                 preferred_element_type=jnp.float32)
        mx = jnp.max(logits, axis=-1, keepdims=True)
        p = jnp.exp(logits - mx)
        g = p / jnp.sum(p, axis=-1, keepdims=True)
        ii = lax.broadcasted_iota(jnp.int32, (_T, _E), 1)
        m1 = jnp.max(g, axis=-1, keepdims=True)
        i1 = jnp.min(jnp.where(g == m1, ii, _E), axis=-1, keepdims=True)
        sel1 = ii == i1
        g2 = jnp.where(sel1, -1.0, g)
        m2 = jnp.max(g2, axis=-1, keepdims=True)
        i2 = jnp.min(jnp.where(g2 == m2, ii, _E), axis=-1, keepdims=True)
        ind_ref[...] = jnp.concatenate([i1, i2], axis=1)
        val_ref[...] = jnp.concatenate([m1, m2], axis=1)


def _proj_call(ctx, proj_w, x, ln2_w, ln2_b, gate_w):
    return pl.pallas_call(
        _proj_body,
        grid=(_H,),
        in_specs=[
            pl.BlockSpec((1, _T, _HD), lambda h: (h, 0, 0)),
            pl.BlockSpec((1, _C, _HD), lambda h: (h, 0, 0)),
            pl.BlockSpec((_T, _C), lambda h: (0, 0)),
            pl.BlockSpec((1, _C), lambda h: (0, 0)),
            pl.BlockSpec((1, _C), lambda h: (0, 0)),
            pl.BlockSpec((_E, _C), lambda h: (0, 0)),
        ],
        out_specs=[
            pl.BlockSpec((_T, _C), lambda h: (0, 0)),
            pl.BlockSpec((_T, _C), lambda h: (0, 0)),
            pl.BlockSpec((_T, _TOPK), lambda h: (0, 0)),
            pl.BlockSpec((_T, _TOPK), lambda h: (0, 0)),
        ],
        out_shape=[
            jax.ShapeDtypeStruct((_T, _C), jnp.float32),
            jax.ShapeDtypeStruct((_T, _C), jnp.float32),
            jax.ShapeDtypeStruct((_T, _TOPK), jnp.int32),
            jax.ShapeDtypeStruct((_T, _TOPK), jnp.float32),
        ],
        scratch_shapes=[pltpu.VMEM((_T, _C), jnp.float32)],
    )(ctx, proj_w.reshape(_C, _H, _HD).transpose(1, 0, 2), x,
      ln2_w.reshape(1, _C), ln2_b.reshape(1, _C), gate_w)


_S = _T * _TOPK
_BM = 256
_NB = (_S + _E * _BM) // _BM
_SR, _SL = 32, 128


def _route_body(e32_ref, pos_ref, bexp_ref):
    e32 = e32_ref[...]
    rr = lax.broadcasted_iota(jnp.int32, (_SL, _SL), 0)
    cc = lax.broadcasted_iota(jnp.int32, (_SL, _SL), 1)
    up = (rr < cc).astype(jnp.float32)
    r2 = lax.broadcasted_iota(jnp.int32, (_SR, _SR), 0)
    c2 = lax.broadcasted_iota(jnp.int32, (_SR, _SR), 1)
    lo = (c2 < r2).astype(jnp.float32)

    pos = jnp.zeros((_SR, _SL), jnp.float32)
    start = jnp.float32(0.0)
    cums = []
    for e in range(_E):
        ind = (e32 == e).astype(jnp.float32)
        ex_row = jnp.dot(ind, up, preferred_element_type=jnp.float32,
                         precision=lax.Precision.HIGHEST)
        rowtot = jnp.sum(ind, axis=1, keepdims=True)
        offs = jnp.dot(lo, rowtot, preferred_element_type=jnp.float32,
                       precision=lax.Precision.HIGHEST)
        rank = ex_row + offs
        pos = pos + ind * (start + rank)
        cnt = jnp.sum(ind)
        start = start + jnp.ceil(cnt / _BM) * _BM
        cums.append(start)
    pos_ref[...] = pos.astype(jnp.int32)

    li = lax.broadcasted_iota(jnp.int32, (1, _SL), 1)
    bexp = jnp.zeros((1, _SL), jnp.int32)
    for e in range(_E):
        bexp = bexp + ((li * _BM).astype(jnp.float32) >= cums[e]
                       ).astype(jnp.int32)
    bexp_ref[...] = jnp.minimum(bexp, _E - 1)


def _route_call(e32):
    return pl.pallas_call(
        _route_body,
        out_shape=[
            jax.ShapeDtypeStruct((_SR, _SL), jnp.int32),
            jax.ShapeDtypeStruct((1, _SL), jnp.int32),
        ],
    )(e32)


def _dispatch_body(p0_ref, p1_ref, w0_ref, w1_ref, swt_ref, sid_ref):
    b = pl.program_id(0)
    pr = b * _BM + lax.broadcasted_iota(jnp.int32, (_BM, _T), 0)
    s0 = p0_ref[...] == pr
    s1 = p1_ref[...] == pr
    swt_ref[0] = (
        jnp.sum(jnp.where(s0, w0_ref[...], 0.0), axis=1, keepdims=True)
        + jnp.sum(jnp.where(s1, w1_ref[...], 0.0), axis=1, keepdims=True))
    ti = lax.broadcasted_iota(jnp.int32, (_BM, _T), 1)
    sid_ref[0] = (jnp.sum(jnp.where(s0, ti, 0), axis=1, keepdims=True)
                  + jnp.sum(jnp.where(s1, ti, 0), axis=1, keepdims=True))


def _dispatch_call(p0_row, p1_row, w0_row, w1_row):
    return pl.pallas_call(
        _dispatch_body,
        grid=(_NB,),
        in_specs=[
            pl.BlockSpec((1, _T), lambda b: (0, 0)),
            pl.BlockSpec((1, _T), lambda b: (0, 0)),
            pl.BlockSpec((1, _T), lambda b: (0, 0)),
            pl.BlockSpec((1, _T), lambda b: (0, 0)),
        ],
        out_specs=[
            pl.BlockSpec((1, _BM, 1), lambda b: (b, 0, 0)),
            pl.BlockSpec((1, _BM, 1), lambda b: (b, 0, 0)),
        ],
        out_shape=[
            jax.ShapeDtypeStruct((_NB, _BM, 1), jnp.float32),
            jax.ShapeDtypeStruct((_NB, _BM, 1), jnp.int32),
        ],
    )(p0_row, p1_row, w0_row, w1_row)


_P = _NB * _BM
_NWK = 32
_GCH = 64


def _sc_gather_call(h2, sids):
    from jax.experimental.pallas import tpu_sc as plsc
    mesh = plsc.VectorSubcoreMesh(core_axis_name="c", subcore_axis_name="s")
    b_per_w = _P // _NWK

    @functools.partial(
        pl.kernel, mesh=mesh,
        out_type=jax.ShapeDtypeStruct((_P, _C), jnp.float32),
        scratch_types=[
            pltpu.VMEM((_GCH,), jnp.int32),
            pltpu.VMEM((_GCH, _C), jnp.float32),
            pltpu.SemaphoreType.DMA,
        ],
    )
    def k(h2_hbm, sids_hbm, xd_hbm, idx_v, rows_v, sem):
        wid = lax.axis_index("s") * 2 + lax.axis_index("c")
        for j in range(b_per_w // _GCH):
            base = wid * b_per_w + j * _GCH
            pltpu.sync_copy(sids_hbm.at[pl.ds(base, _GCH)], idx_v)
            pltpu.async_copy(h2_hbm.at[idx_v], rows_v, sem).wait()
            pltpu.sync_copy(rows_v, xd_hbm.at[pl.ds(base, _GCH)])

    return k(h2, sids)


_NF = 4
_FH = _FF // _NF


def _ffn_body(bexp_ref, xd_ref, w1_ref, b1_ref, w2_ref, b2_ref, swt_ref,
              od_ref, oacc_ref):
    f = pl.program_id(0)
    b = pl.program_id(1)
    dn = (((1,), (1,)), ((), ()))
    hf = lax.dot_general(xd_ref[0], w1_ref[0, 0], dn,
                         preferred_element_type=jnp.float32)
    hf = _gelu_exact(hf + b1_ref[0, 0, 0])
    o = lax.dot_general(hf, w2_ref[0], dn,
                        preferred_element_type=jnp.float32)

    @pl.when(f == 0)
    def _():
        oacc_ref[b] = o

    @pl.when(jnp.logical_and(f > 0, f < _NF - 1))
    def _():
        oacc_ref[b] += o

    @pl.when(f == _NF - 1)
    def _():
        prev = oacc_ref[b] if _NF > 1 else jnp.zeros_like(o)
        od_ref[0] = (prev + o + b2_ref[0, 0]) * swt_ref[0]


def _ffn_call(bexp, xd, w1, b1, w2, b2, swt):
    grid_spec = pltpu.PrefetchScalarGridSpec(
        num_scalar_prefetch=1,
        grid=(_NF, _NB),
        in_specs=[
            pl.BlockSpec((1, _BM, _C), lambda f, b, be: (b, 0, 0)),
            pl.BlockSpec((1, 1, _FH, _C), lambda f, b, be: (be[b], f, 0, 0)),
            pl.BlockSpec((1, 1, 1, _FH), lambda f, b, be: (be[b], f, 0, 0)),
            pl.BlockSpec((1, _C, _FH), lambda f, b, be: (be[b], 0, f)),
            pl.BlockSpec((1, 1, _C), lambda f, b, be: (be[b], 0, 0)),
            pl.BlockSpec((1, _BM, 1), lambda f, b, be: (b, 0, 0)),
        ],
        out_specs=pl.BlockSpec((1, _BM, _C), lambda f, b, be: (b, 0, 0)),
        scratch_shapes=[pltpu.VMEM((_NB, _BM, _C), jnp.float32)],
    )
    return pl.pallas_call(
        _ffn_body,
        grid_spec=grid_spec,
        out_shape=jax.ShapeDtypeStruct((_NB, _BM, _C), jnp.float32),
    )(bexp, xd, w1.reshape(_E, _NF, _FH, _C), b1.reshape(_E, _NF, 1, _FH),
      w2, b2.reshape(_E, 1, _C), swt)


def _combine_body(od_ref, p0_ref, p1_ref, x1_ref, out_ref):
    b = pl.program_id(0)

    @pl.when(b == 0)
    def _():
        out_ref[...] = x1_ref[...]

    pc = b * _BM + lax.broadcasted_iota(jnp.int32, (_T, _BM), 1)
    selT = ((p0_ref[...] == pc).astype(jnp.float32)
            + (p1_ref[...] == pc).astype(jnp.float32))
    out_ref[...] += jnp.dot(selT, od_ref[0],
                            preferred_element_type=jnp.float32)


def _combine_call(od, p0_col, p1_col, x1):
    return pl.pallas_call(
        _combine_body,
        grid=(_NB,),
        in_specs=[
            pl.BlockSpec((1, _BM, _C), lambda b: (b, 0, 0)),
            pl.BlockSpec((_T, 1), lambda b: (0, 0)),
            pl.BlockSpec((_T, 1), lambda b: (0, 0)),
            pl.BlockSpec((_T, _C), lambda b: (0, 0)),
        ],
        out_specs=pl.BlockSpec((_T, _C), lambda b: (0, 0)),
        out_shape=jax.ShapeDtypeStruct((_T, _C), jnp.float32),
    )(od, p0_col, p1_col, x1)


def kernel(x, ln1_w, ln1_b, ln2_w, ln2_b, qkv_w, proj_w, gate_w, w1, b1,
           w2, b2):
    x2d = x.reshape(_T, _C)
    pos = jnp.arange(_T, dtype=jnp.float32)[:, None]
    inv_freq = 1.0 / (10000.0 ** (
        jnp.arange(0, _HD, 2, dtype=jnp.float32) / _HD))
    ang = pos * inv_freq
    sin = jnp.sin(ang)
    cos = jnp.cos(ang)

    ln1 = _ln_call(x2d, ln1_w, ln1_b)
    q, k, v = _qkv_call(ln1, qkv_w, sin, cos)
    ctx = _attn_call(q, k, v)
    x1, h2, inds2, vals2 = _proj_call(ctx, proj_w, x2d, ln2_w, ln2_b, gate_w)

    e32 = inds2.reshape(_SR, _SL)
    pos32, bexp2d = _route_call(e32)
    pos2 = pos32.reshape(_T, _TOPK)
    p0_col, p1_col = pos2[:, 0:1], pos2[:, 1:2]
    p0_row, p1_row = p0_col.reshape(1, _T), p1_col.reshape(1, _T)
    w0_row = vals2[:, 0].reshape(1, _T)
    w1_row = vals2[:, 1].reshape(1, _T)
    bexp = bexp2d.reshape(_SL)[:_NB]

    swt, sid = _dispatch_call(p0_row, p1_row, w0_row, w1_row)
    xd = _sc_gather_call(h2, sid.reshape(_P)).reshape(_NB, _BM, _C)
    od = _ffn_call(bexp, xd, w1, b1, w2, b2, swt)
    out = _combine_call(od, p0_col, p1_col, x1)
    return out.reshape(_B, _T, _C)


# speedup vs baseline: 1.2049x; 1.2049x over previous
import functools
import math

import jax
import jax.numpy as jnp
from jax import lax
from jax.experimental import pallas as pl
from jax.experimental.pallas import tpu as pltpu

_B, _T, _C = 1, 2048, 1024
_H, _HD = 16, 64
_E, _TOPK, _FF = 8, 2, 4096
_HALF = _HD // 2
_SQRT2 = math.sqrt(2.0)


def _layernorm(x, w, b, eps=1e-5, clip=65000.0):
    mu = jnp.mean(x, axis=-1, keepdims=True)
    xc = x - mu
    var = jnp.mean(xc * xc, axis=-1, keepdims=True)
    y = xc * lax.rsqrt(var + eps)
    y = jnp.clip(y, -clip, clip)
    return y * w + b


def _gelu_exact(x):
    return 0.5 * x * (1.0 + lax.erf(x / _SQRT2))


def _ln_body(x_ref, w_ref, b_ref, o_ref):
    o_ref[...] = _layernorm(x_ref[...], w_ref[...], b_ref[...])


def _ln_call(x, w, b):
    return pl.pallas_call(
        _ln_body,
        out_shape=jax.ShapeDtypeStruct((_T, _C), jnp.float32),
    )(x, w.reshape(1, _C), b.reshape(1, _C))


def _qkv_body(ln_ref, w_ref, sin_ref, cos_ref, q_ref, k_ref, v_ref):
    x = ln_ref[...]
    dn = (((1,), (1,)), ((), ()))
    y = lax.dot_general(x, w_ref[0], dn,
                        preferred_element_type=jnp.float32)
    sin = sin_ref[...]
    cos = cos_ref[...]

    def rope(t):
        t1 = t[:, :_HALF]
        t2 = t[:, _HALF:]
        return jnp.concatenate([t1 * cos - t2 * sin, t1 * sin + t2 * cos],
                               axis=-1)

    q_ref[0] = rope(y[:, :_HD])
    k_ref[0] = rope(y[:, _HD:2 * _HD])
    v_ref[0] = y[:, 2 * _HD:]


def _qkv_call(ln1, qkv_w, sin, cos):
    return pl.pallas_call(
        _qkv_body,
        grid=(_H,),
        in_specs=[
            pl.BlockSpec((_T, _C), lambda h: (0, 0)),
            pl.BlockSpec((1, 3 * _HD, _C), lambda h: (h, 0, 0)),
            pl.BlockSpec((_T, _HALF), lambda h: (0, 0)),
            pl.BlockSpec((_T, _HALF), lambda h: (0, 0)),
        ],
        out_specs=[
            pl.BlockSpec((1, _T, _HD), lambda h: (h, 0, 0)),
            pl.BlockSpec((1, _T, _HD), lambda h: (h, 0, 0)),
            pl.BlockSpec((1, _T, _HD), lambda h: (h, 0, 0)),
        ],
        out_shape=[jax.ShapeDtypeStruct((_H, _T, _HD), jnp.float32)] * 3,
    )(ln1, qkv_w.reshape(_H, 3 * _HD, _C), sin, cos)


_BQ = 512
_BK = 256


def _attn_body(q_ref, k_ref, v_ref, o_ref):
    qb = pl.program_id(1)
    q = q_ref[0]
    scale = 1.0 / math.sqrt(_HD)
    dn = (((1,), (1,)), ((), ()))

    def process(klen):
        s = lax.dot_general(q, k_ref[0, :klen, :], dn,
                            preferred_element_type=jnp.float32)
        s = s * scale
        col = lax.broadcasted_iota(jnp.int32, (_BQ, klen), 1)
        row = qb * _BQ + lax.broadcasted_iota(jnp.int32, (_BQ, klen), 0)
        s = jnp.where(col <= row, s, -1e30)
        m = jnp.max(s, axis=-1, keepdims=True)
        p = jnp.exp(s - m)
        l = jnp.sum(p, axis=-1, keepdims=True)
        ctx = lax.dot_general(
            p, v_ref[0, :klen, :], (((1,), (0,)), ((), ())),
            preferred_element_type=jnp.float32)
        o_ref[0] = ctx / l

    nspan = 1
    for i in range(_T // _BQ // nspan):
        @pl.when(qb // nspan == i)
        def _():
            process((i + 1) * nspan * _BQ)


def _attn_call(q, k, v):
    return pl.pallas_call(
        _attn_body,
        grid=(_H, _T // _BQ),
        in_specs=[
            pl.BlockSpec((1, _BQ, _HD), lambda h, qb: (h, qb, 0)),
            pl.BlockSpec((1, _T, _HD), lambda h, qb: (h, 0, 0)),
            pl.BlockSpec((1, _T, _HD), lambda h, qb: (h, 0, 0)),
        ],
        out_specs=pl.BlockSpec((1, _BQ, _HD), lambda h, qb: (h, qb, 0)),
        out_shape=jax.ShapeDtypeStruct((_H, _T, _HD), jnp.float32),
    )(q, k, v)


def _proj_body(ctx_ref, pw_ref, x_ref, w2_ref, b2_ref, gw_ref,
               x1_ref, h2_ref, ind_ref, val_ref, acc_ref):
    h = pl.program_id(0)

    @pl.when(h == 0)
    def _():
        acc_ref[...] = jnp.zeros_like(acc_ref)

    dn = (((1,), (1,)), ((), ()))
    acc_ref[...] += lax.dot_general(ctx_ref[0], pw_ref[0], dn,
                                    preferred_element_type=jnp.float32)

    @pl.when(h == _H - 1)
    def _():
        x1 = x_ref[...] + acc_ref[...]
        x1_ref[...] = x1
        h2 = _layernorm(x1, w2_ref[...], b2_ref[...])
        h2_ref[...] = h2
        logits = lax.dot_general(h2, gw_ref[...], dn,
                                 preferred_element_type=jnp.float32)
        mx = jnp.max(logits, axis=-1, keepdims=True)
        p = jnp.exp(logits - mx)
        g = p / jnp.sum(p, axis=-1, keepdims=True)
        ii = lax.broadcasted_iota(jnp.int32, (_T, _E), 1)
        m1 = jnp.max(g, axis=-1, keepdims=True)
        i1 = jnp.min(jnp.where(g == m1, ii, _E), axis=-1, keepdims=True)
        sel1 = ii == i1
        g2 = jnp.where(sel1, -1.0, g)
        m2 = jnp.max(g2, axis=-1, keepdims=True)
        i2 = jnp.min(jnp.where(g2 == m2, ii, _E), axis=-1, keepdims=True)
        ind_ref[...] = jnp.concatenate([i1, i2], axis=1)
        val_ref[...] = jnp.concatenate([m1, m2], axis=1)


def _proj_call(ctx, proj_w, x, ln2_w, ln2_b, gate_w):
    return pl.pallas_call(
        _proj_body,
        grid=(_H,),
        in_specs=[
            pl.BlockSpec((1, _T, _HD), lambda h: (h, 0, 0)),
            pl.BlockSpec((1, _C, _HD), lambda h: (h, 0, 0)),
            pl.BlockSpec((_T, _C), lambda h: (0, 0)),
            pl.BlockSpec((1, _C), lambda h: (0, 0)),
            pl.BlockSpec((1, _C), lambda h: (0, 0)),
            pl.BlockSpec((_E, _C), lambda h: (0, 0)),
        ],
        out_specs=[
            pl.BlockSpec((_T, _C), lambda h: (0, 0)),
            pl.BlockSpec((_T, _C), lambda h: (0, 0)),
            pl.BlockSpec((_T, _TOPK), lambda h: (0, 0)),
            pl.BlockSpec((_T, _TOPK), lambda h: (0, 0)),
        ],
        out_shape=[
            jax.ShapeDtypeStruct((_T, _C), jnp.float32),
            jax.ShapeDtypeStruct((_T, _C), jnp.float32),
            jax.ShapeDtypeStruct((_T, _TOPK), jnp.int32),
            jax.ShapeDtypeStruct((_T, _TOPK), jnp.float32),
        ],
        scratch_shapes=[pltpu.VMEM((_T, _C), jnp.float32)],
    )(ctx, proj_w.reshape(_C, _H, _HD).transpose(1, 0, 2), x,
      ln2_w.reshape(1, _C), ln2_b.reshape(1, _C), gate_w)


_S = _T * _TOPK
_BM = 256
_NB = (_S + _E * _BM) // _BM
_SR, _SL = 32, 128


def _route_body(e32_ref, pos_ref, bexp_ref):
    e32 = e32_ref[...]
    rr = lax.broadcasted_iota(jnp.int32, (_SL, _SL), 0)
    cc = lax.broadcasted_iota(jnp.int32, (_SL, _SL), 1)
    up = (rr < cc).astype(jnp.float32)
    r2 = lax.broadcasted_iota(jnp.int32, (_SR, _SR), 0)
    c2 = lax.broadcasted_iota(jnp.int32, (_SR, _SR), 1)
    lo = (c2 < r2).astype(jnp.float32)

    pos = jnp.zeros((_SR, _SL), jnp.float32)
    start = jnp.float32(0.0)
    cums = []
    for e in range(_E):
        ind = (e32 == e).astype(jnp.float32)
        ex_row = jnp.dot(ind, up, preferred_element_type=jnp.float32,
                         precision=lax.Precision.HIGHEST)
        rowtot = jnp.sum(ind, axis=1, keepdims=True)
        offs = jnp.dot(lo, rowtot, preferred_element_type=jnp.float32,
                       precision=lax.Precision.HIGHEST)
        rank = ex_row + offs
        pos = pos + ind * (start + rank)
        cnt = jnp.sum(ind)
        start = start + jnp.ceil(cnt / _BM) * _BM
        cums.append(start)
    pos_ref[...] = pos.astype(jnp.int32)

    li = lax.broadcasted_iota(jnp.int32, (1, _SL), 1)
    bexp = jnp.zeros((1, _SL), jnp.int32)
    for e in range(_E):
        bexp = bexp + ((li * _BM).astype(jnp.float32) >= cums[e]
                       ).astype(jnp.int32)
    bexp_ref[...] = jnp.minimum(bexp, _E - 1)


def _route_call(e32):
    return pl.pallas_call(
        _route_body,
        out_shape=[
            jax.ShapeDtypeStruct((_SR, _SL), jnp.int32),
            jax.ShapeDtypeStruct((1, _SL), jnp.int32),
        ],
    )(e32)


def _dispatch_body(p0_ref, p1_ref, w0_ref, w1_ref, h2_ref,
                   swt_ref, xd_ref):
    b = pl.program_id(0)
    pr = b * _BM + lax.broadcasted_iota(jnp.int32, (_BM, _T), 0)
    s0 = p0_ref[...] == pr
    s1 = p1_ref[...] == pr
    sel = s0.astype(jnp.float32) + s1.astype(jnp.float32)
    swt_ref[0] = (
        jnp.sum(jnp.where(s0, w0_ref[...], 0.0), axis=1, keepdims=True)
        + jnp.sum(jnp.where(s1, w1_ref[...], 0.0), axis=1, keepdims=True))
    xd_ref[0] = jnp.dot(sel, h2_ref[...],
                        preferred_element_type=jnp.float32)


def _dispatch_call(p0_row, p1_row, w0_row, w1_row, h2):
    return pl.pallas_call(
        _dispatch_body,
        grid=(_NB,),
        in_specs=[
            pl.BlockSpec((1, _T), lambda b: (0, 0)),
            pl.BlockSpec((1, _T), lambda b: (0, 0)),
            pl.BlockSpec((1, _T), lambda b: (0, 0)),
            pl.BlockSpec((1, _T), lambda b: (0, 0)),
            pl.BlockSpec((_T, _C), lambda b: (0, 0)),
        ],
        out_specs=[
            pl.BlockSpec((1, _BM, 1), lambda b: (b, 0, 0)),
            pl.BlockSpec((1, _BM, _C), lambda b: (b, 0, 0)),
        ],
        out_shape=[
            jax.ShapeDtypeStruct((_NB, _BM, 1), jnp.float32),
            jax.ShapeDtypeStruct((_NB, _BM, _C), jnp.float32),
        ],
    )(p0_row, p1_row, w0_row, w1_row, h2)


_NF = 4
_FH = _FF // _NF


def _ffn_body(bexp_ref, xd_ref, w1_ref, b1_ref, w2_ref, b2_ref, swt_ref,
              od_ref, oacc_ref):
    f = pl.program_id(0)
    b = pl.program_id(1)
    dn = (((1,), (1,)), ((), ()))
    hf = lax.dot_general(xd_ref[0], w1_ref[0, 0], dn,
                         preferred_element_type=jnp.float32)
    hf = _gelu_exact(hf + b1_ref[0, 0, 0])
    o = lax.dot_general(hf, w2_ref[0], dn,
                        preferred_element_type=jnp.float32)

    @pl.when(f == 0)
    def _():
        oacc_ref[b] = o

    @pl.when(jnp.logical_and(f > 0, f < _NF - 1))
    def _():
        oacc_ref[b] += o

    @pl.when(f == _NF - 1)
    def _():
        prev = oacc_ref[b] if _NF > 1 else jnp.zeros_like(o)
        od_ref[0] = (prev + o + b2_ref[0, 0]) * swt_ref[0]


def _ffn_call(bexp, xd, w1, b1, w2, b2, swt):
    grid_spec = pltpu.PrefetchScalarGridSpec(
        num_scalar_prefetch=1,
        grid=(_NF, _NB),
        in_specs=[
            pl.BlockSpec((1, _BM, _C), lambda f, b, be: (b, 0, 0)),
            pl.BlockSpec((1, 1, _FH, _C), lambda f, b, be: (be[b], f, 0, 0)),
            pl.BlockSpec((1, 1, 1, _FH), lambda f, b, be: (be[b], f, 0, 0)),
            pl.BlockSpec((1, _C, _FH), lambda f, b, be: (be[b], 0, f)),
            pl.BlockSpec((1, 1, _C), lambda f, b, be: (be[b], 0, 0)),
            pl.BlockSpec((1, _BM, 1), lambda f, b, be: (b, 0, 0)),
        ],
        out_specs=pl.BlockSpec((1, _BM, _C), lambda f, b, be: (b, 0, 0)),
        scratch_shapes=[pltpu.VMEM((_NB, _BM, _C), jnp.float32)],
    )
    return pl.pallas_call(
        _ffn_body,
        grid_spec=grid_spec,
        out_shape=jax.ShapeDtypeStruct((_NB, _BM, _C), jnp.float32),
    )(bexp, xd, w1.reshape(_E, _NF, _FH, _C), b1.reshape(_E, _NF, 1, _FH),
      w2, b2.reshape(_E, 1, _C), swt)


def _combine_body(od_ref, p0_ref, p1_ref, x1_ref, out_ref):
    b = pl.program_id(0)

    @pl.when(b == 0)
    def _():
        out_ref[...] = x1_ref[...]

    pc = b * _BM + lax.broadcasted_iota(jnp.int32, (_T, _BM), 1)
    selT = ((p0_ref[...] == pc).astype(jnp.float32)
            + (p1_ref[...] == pc).astype(jnp.float32))
    out_ref[...] += jnp.dot(selT, od_ref[0],
                            preferred_element_type=jnp.float32)


def _combine_call(od, p0_col, p1_col, x1):
    return pl.pallas_call(
        _combine_body,
        grid=(_NB,),
        in_specs=[
            pl.BlockSpec((1, _BM, _C), lambda b: (b, 0, 0)),
            pl.BlockSpec((_T, 1), lambda b: (0, 0)),
            pl.BlockSpec((_T, 1), lambda b: (0, 0)),
            pl.BlockSpec((_T, _C), lambda b: (0, 0)),
        ],
        out_specs=pl.BlockSpec((_T, _C), lambda b: (0, 0)),
        out_shape=jax.ShapeDtypeStruct((_T, _C), jnp.float32),
    )(od, p0_col, p1_col, x1)


def kernel(x, ln1_w, ln1_b, ln2_w, ln2_b, qkv_w, proj_w, gate_w, w1, b1,
           w2, b2):
    x2d = x.reshape(_T, _C)
    pos = jnp.arange(_T, dtype=jnp.float32)[:, None]
    inv_freq = 1.0 / (10000.0 ** (
        jnp.arange(0, _HD, 2, dtype=jnp.float32) / _HD))
    ang = pos * inv_freq
    sin = jnp.sin(ang)
    cos = jnp.cos(ang)

    ln1 = _ln_call(x2d, ln1_w, ln1_b)
    q, k, v = _qkv_call(ln1, qkv_w, sin, cos)
    ctx = _attn_call(q, k, v)
    x1, h2, inds2, vals2 = _proj_call(ctx, proj_w, x2d, ln2_w, ln2_b, gate_w)

    e32 = inds2.reshape(_SR, _SL)
    pos32, bexp2d = _route_call(e32)
    pos2 = pos32.reshape(_T, _TOPK)
    p0_col, p1_col = pos2[:, 0:1], pos2[:, 1:2]
    p0_row, p1_row = p0_col.reshape(1, _T), p1_col.reshape(1, _T)
    w0_row = vals2[:, 0].reshape(1, _T)
    w1_row = vals2[:, 1].reshape(1, _T)
    bexp = bexp2d.reshape(_SL)[:_NB]

    swt, xd = _dispatch_call(p0_row, p1_row, w0_row, w1_row, h2)
    od = _ffn_call(bexp, xd, w1, b1, w2, b2, swt)
    out = _combine_call(od, p0_col, p1_col, x1)
    return out.reshape(_B, _T, _C)


# speedup vs baseline: 1.2188x; 1.0115x over previous
import functools
import math

import jax
import jax.numpy as jnp
from jax import lax
from jax.experimental import pallas as pl
from jax.experimental.pallas import tpu as pltpu

_B, _T, _C = 1, 2048, 1024
_H, _HD = 16, 64
_E, _TOPK, _FF = 8, 2, 4096
_HALF = _HD // 2
_SQRT2 = math.sqrt(2.0)


def _layernorm(x, w, b, eps=1e-5, clip=65000.0):
    mu = jnp.mean(x, axis=-1, keepdims=True)
    xc = x - mu
    var = jnp.mean(xc * xc, axis=-1, keepdims=True)
    y = xc * lax.rsqrt(var + eps)
    y = jnp.clip(y, -clip, clip)
    return y * w + b


def _gelu_exact(x):
    return 0.5 * x * (1.0 + lax.erf(x / _SQRT2))


def _ln_body(x_ref, w_ref, b_ref, o_ref):
    o_ref[...] = _layernorm(x_ref[...], w_ref[...], b_ref[...])


def _ln_call(x, w, b):
    return pl.pallas_call(
        _ln_body,
        out_shape=jax.ShapeDtypeStruct((_T, _C), jnp.float32),
    )(x, w.reshape(1, _C), b.reshape(1, _C))


def _qkv_body(ln_ref, w_ref, sin_ref, cos_ref, q_ref, k_ref, v_ref):
    x = ln_ref[...]
    dn = (((1,), (1,)), ((), ()))
    y = lax.dot_general(x, w_ref[0], dn,
                        preferred_element_type=jnp.float32)
    sin = sin_ref[...]
    cos = cos_ref[...]

    def rope(t):
        t1 = t[:, :_HALF]
        t2 = t[:, _HALF:]
        return jnp.concatenate([t1 * cos - t2 * sin, t1 * sin + t2 * cos],
                               axis=-1)

    q_ref[0] = rope(y[:, :_HD])
    k_ref[0] = rope(y[:, _HD:2 * _HD])
    v_ref[0] = y[:, 2 * _HD:]


def _qkv_call(ln1, qkv_w, sin, cos):
    return pl.pallas_call(
        _qkv_body,
        grid=(_H,),
        in_specs=[
            pl.BlockSpec((_T, _C), lambda h: (0, 0)),
            pl.BlockSpec((1, 3 * _HD, _C), lambda h: (h, 0, 0)),
            pl.BlockSpec((_T, _HALF), lambda h: (0, 0)),
            pl.BlockSpec((_T, _HALF), lambda h: (0, 0)),
        ],
        out_specs=[
            pl.BlockSpec((1, _T, _HD), lambda h: (h, 0, 0)),
            pl.BlockSpec((1, _T, _HD), lambda h: (h, 0, 0)),
            pl.BlockSpec((1, _T, _HD), lambda h: (h, 0, 0)),
        ],
        out_shape=[jax.ShapeDtypeStruct((_H, _T, _HD), jnp.float32)] * 3,
    )(ln1, qkv_w.reshape(_H, 3 * _HD, _C), sin, cos)


_BQ = 512
_BK = 256


def _attn_body(q_ref, k_ref, v_ref, o_ref):
    qb = pl.program_id(1)
    q = q_ref[0]
    scale = 1.0 / math.sqrt(_HD)
    dn = (((1,), (1,)), ((), ()))

    def process(klen):
        s = lax.dot_general(q, k_ref[0, :klen, :], dn,
                            preferred_element_type=jnp.float32)
        s = s * scale
        col = lax.broadcasted_iota(jnp.int32, (_BQ, klen), 1)
        row = qb * _BQ + lax.broadcasted_iota(jnp.int32, (_BQ, klen), 0)
        s = jnp.where(col <= row, s, -1e30)
        m = jnp.max(s, axis=-1, keepdims=True)
        p = jnp.exp(s - m)
        l = jnp.sum(p, axis=-1, keepdims=True)
        ctx = lax.dot_general(
            p, v_ref[0, :klen, :], (((1,), (0,)), ((), ())),
            preferred_element_type=jnp.float32)
        o_ref[0] = ctx / l

    nspan = 1
    for i in range(_T // _BQ // nspan):
        @pl.when(qb // nspan == i)
        def _():
            process((i + 1) * nspan * _BQ)


def _attn_call(q, k, v):
    return pl.pallas_call(
        _attn_body,
        grid=(_H, _T // _BQ),
        in_specs=[
            pl.BlockSpec((1, _BQ, _HD), lambda h, qb: (h, qb, 0)),
            pl.BlockSpec((1, _T, _HD), lambda h, qb: (h, 0, 0)),
            pl.BlockSpec((1, _T, _HD), lambda h, qb: (h, 0, 0)),
        ],
        out_specs=pl.BlockSpec((1, _BQ, _HD), lambda h, qb: (h, qb, 0)),
        out_shape=jax.ShapeDtypeStruct((_H, _T, _HD), jnp.float32),
    )(q, k, v)


def _proj_body(ctx_ref, pw_ref, x_ref, w2_ref, b2_ref, gw_ref,
               x1_ref, h2_ref, ind_ref, val_ref, acc_ref):
    h = pl.program_id(0)

    @pl.when(h == 0)
    def _():
        acc_ref[...] = jnp.zeros_like(acc_ref)

    dn = (((1,), (1,)), ((), ()))
    acc_ref[...] += lax.dot_general(ctx_ref[0], pw_ref[0], dn,
                                    preferred_element_type=jnp.float32)

    @pl.when(h == _H - 1)
    def _():
        x1 = x_ref[...] + acc_ref[...]
        x1_ref[...] = x1
        h2 = _layernorm(x1, w2_ref[...], b2_ref[...])
        h2_ref[...] = h2
        logits = lax.dot_general(h2, gw_ref[...], dn,
                                 preferred_element_type=jnp.float32)
        mx = jnp.max(logits, axis=-1, keepdims=True)
        p = jnp.exp(logits - mx)
        g = p / jnp.sum(p, axis=-1, keepdims=True)
        ii = lax.broadcasted_iota(jnp.int32, (_T, _E), 1)
        m1 = jnp.max(g, axis=-1, keepdims=True)
        i1 = jnp.min(jnp.where(g == m1, ii, _E), axis=-1, keepdims=True)
        sel1 = ii == i1
        g2 = jnp.where(sel1, -1.0, g)
        m2 = jnp.max(g2, axis=-1, keepdims=True)
        i2 = jnp.min(jnp.where(g2 == m2, ii, _E), axis=-1, keepdims=True)
        ind_ref[...] = jnp.concatenate([i1, i2], axis=1)
        val_ref[...] = jnp.concatenate([m1, m2], axis=1)


def _proj_call(ctx, proj_w, x, ln2_w, ln2_b, gate_w):
    return pl.pallas_call(
        _proj_body,
        grid=(_H,),
        in_specs=[
            pl.BlockSpec((1, _T, _HD), lambda h: (h, 0, 0)),
            pl.BlockSpec((1, _C, _HD), lambda h: (h, 0, 0)),
            pl.BlockSpec((_T, _C), lambda h: (0, 0)),
            pl.BlockSpec((1, _C), lambda h: (0, 0)),
            pl.BlockSpec((1, _C), lambda h: (0, 0)),
            pl.BlockSpec((_E, _C), lambda h: (0, 0)),
        ],
        out_specs=[
            pl.BlockSpec((_T, _C), lambda h: (0, 0)),
            pl.BlockSpec((_T, _C), lambda h: (0, 0)),
            pl.BlockSpec((_T, _TOPK), lambda h: (0, 0)),
            pl.BlockSpec((_T, _TOPK), lambda h: (0, 0)),
        ],
        out_shape=[
            jax.ShapeDtypeStruct((_T, _C), jnp.float32),
            jax.ShapeDtypeStruct((_T, _C), jnp.float32),
            jax.ShapeDtypeStruct((_T, _TOPK), jnp.int32),
            jax.ShapeDtypeStruct((_T, _TOPK), jnp.float32),
        ],
        scratch_shapes=[pltpu.VMEM((_T, _C), jnp.float32)],
    )(ctx, proj_w.reshape(_C, _H, _HD).transpose(1, 0, 2), x,
      ln2_w.reshape(1, _C), ln2_b.reshape(1, _C), gate_w)


_S = _T * _TOPK
_BM = 256
_NB = (_S + _E * _BM) // _BM
_SR, _SL = 32, 128


def _route_body(e32_ref, pos_ref, bexp_ref):
    e32 = e32_ref[...]
    rr = lax.broadcasted_iota(jnp.int32, (_SL, _SL), 0)
    cc = lax.broadcasted_iota(jnp.int32, (_SL, _SL), 1)
    up = (rr < cc).astype(jnp.float32)
    r2 = lax.broadcasted_iota(jnp.int32, (_SR, _SR), 0)
    c2 = lax.broadcasted_iota(jnp.int32, (_SR, _SR), 1)
    lo = (c2 < r2).astype(jnp.float32)

    pos = jnp.zeros((_SR, _SL), jnp.float32)
    start = jnp.float32(0.0)
    cums = []
    for e in range(_E):
        ind = (e32 == e).astype(jnp.float32)
        ex_row = jnp.dot(ind, up, preferred_element_type=jnp.float32,
                         precision=lax.Precision.HIGHEST)
        rowtot = jnp.sum(ind, axis=1, keepdims=True)
        offs = jnp.dot(lo, rowtot, preferred_element_type=jnp.float32,
                       precision=lax.Precision.HIGHEST)
        rank = ex_row + offs
        pos = pos + ind * (start + rank)
        cnt = jnp.sum(ind)
        start = start + jnp.ceil(cnt / _BM) * _BM
        cums.append(start)
    pos_ref[...] = pos.astype(jnp.int32)

    li = lax.broadcasted_iota(jnp.int32, (1, _SL), 1)
    bexp = jnp.zeros((1, _SL), jnp.int32)
    for e in range(_E):
        bexp = bexp + ((li * _BM).astype(jnp.float32) >= cums[e]
                       ).astype(jnp.int32)
    bexp_ref[...] = jnp.minimum(bexp, _E - 1)


def _route_call(e32):
    return pl.pallas_call(
        _route_body,
        out_shape=[
            jax.ShapeDtypeStruct((_SR, _SL), jnp.int32),
            jax.ShapeDtypeStruct((1, _SL), jnp.int32),
        ],
    )(e32)


def _dispatch_body(p0_ref, p1_ref, w0_ref, w1_ref, h2_ref,
                   swt_ref, xd_ref):
    b = pl.program_id(0)
    pr = b * _BM + lax.broadcasted_iota(jnp.int32, (_BM, _T), 0)
    s0 = p0_ref[...] == pr
    s1 = p1_ref[...] == pr
    sel = s0.astype(jnp.bfloat16) + s1.astype(jnp.bfloat16)
    swt_ref[0] = (
        jnp.sum(jnp.where(s0, w0_ref[...], 0.0), axis=1, keepdims=True)
        + jnp.sum(jnp.where(s1, w1_ref[...], 0.0), axis=1, keepdims=True))
    xd_ref[0] = jnp.dot(sel, h2_ref[...],
                        preferred_element_type=jnp.float32)


def _dispatch_call(p0_row, p1_row, w0_row, w1_row, h2):
    return pl.pallas_call(
        _dispatch_body,
        grid=(_NB,),
        in_specs=[
            pl.BlockSpec((1, _T), lambda b: (0, 0)),
            pl.BlockSpec((1, _T), lambda b: (0, 0)),
            pl.BlockSpec((1, _T), lambda b: (0, 0)),
            pl.BlockSpec((1, _T), lambda b: (0, 0)),
            pl.BlockSpec((_T, _C), lambda b: (0, 0)),
        ],
        out_specs=[
            pl.BlockSpec((1, _BM, 1), lambda b: (b, 0, 0)),
            pl.BlockSpec((1, _BM, _C), lambda b: (b, 0, 0)),
        ],
        out_shape=[
            jax.ShapeDtypeStruct((_NB, _BM, 1), jnp.float32),
            jax.ShapeDtypeStruct((_NB, _BM, _C), jnp.float32),
        ],
    )(p0_row, p1_row, w0_row, w1_row, h2)


_NF = 4
_FH = _FF // _NF


def _ffn_body(bexp_ref, xd_ref, w1_ref, b1_ref, w2_ref, b2_ref, swt_ref,
              od_ref, oacc_ref):
    f = pl.program_id(0)
    b = pl.program_id(1)
    dn = (((1,), (1,)), ((), ()))
    hf = lax.dot_general(xd_ref[0], w1_ref[0, 0], dn,
                         preferred_element_type=jnp.float32)
    hf = _gelu_exact(hf + b1_ref[0, 0, 0])
    o = lax.dot_general(hf, w2_ref[0], dn,
                        preferred_element_type=jnp.float32)

    @pl.when(f == 0)
    def _():
        oacc_ref[b] = o

    @pl.when(jnp.logical_and(f > 0, f < _NF - 1))
    def _():
        oacc_ref[b] += o

    @pl.when(f == _NF - 1)
    def _():
        prev = oacc_ref[b] if _NF > 1 else jnp.zeros_like(o)
        od_ref[0] = ((prev + o + b2_ref[0, 0]) * swt_ref[0]
                     ).astype(jnp.bfloat16)


def _ffn_call(bexp, xd, w1, b1, w2, b2, swt):
    grid_spec = pltpu.PrefetchScalarGridSpec(
        num_scalar_prefetch=1,
        grid=(_NF, _NB),
        in_specs=[
            pl.BlockSpec((1, _BM, _C), lambda f, b, be: (b, 0, 0)),
            pl.BlockSpec((1, 1, _FH, _C), lambda f, b, be: (be[b], f, 0, 0)),
            pl.BlockSpec((1, 1, 1, _FH), lambda f, b, be: (be[b], f, 0, 0)),
            pl.BlockSpec((1, _C, _FH), lambda f, b, be: (be[b], 0, f)),
            pl.BlockSpec((1, 1, _C), lambda f, b, be: (be[b], 0, 0)),
            pl.BlockSpec((1, _BM, 1), lambda f, b, be: (b, 0, 0)),
        ],
        out_specs=pl.BlockSpec((1, _BM, _C), lambda f, b, be: (b, 0, 0)),
        scratch_shapes=[pltpu.VMEM((_NB, _BM, _C), jnp.float32)],
    )
    return pl.pallas_call(
        _ffn_body,
        grid_spec=grid_spec,
        out_shape=jax.ShapeDtypeStruct((_NB, _BM, _C), jnp.bfloat16),
    )(bexp, xd, w1.reshape(_E, _NF, _FH, _C), b1.reshape(_E, _NF, 1, _FH),
      w2, b2.reshape(_E, 1, _C), swt)


def _combine_body(od_ref, p0_ref, p1_ref, x1_ref, out_ref):
    b = pl.program_id(0)

    @pl.when(b == 0)
    def _():
        out_ref[...] = x1_ref[...]

    pc = b * _BM + lax.broadcasted_iota(jnp.int32, (_T, _BM), 1)
    selT = ((p0_ref[...] == pc).astype(jnp.bfloat16)
            + (p1_ref[...] == pc).astype(jnp.bfloat16))
    out_ref[...] += jnp.dot(selT, od_ref[0],
                            preferred_element_type=jnp.float32)


def _combine_call(od, p0_col, p1_col, x1):
    return pl.pallas_call(
        _combine_body,
        grid=(_NB,),
        in_specs=[
            pl.BlockSpec((1, _BM, _C), lambda b: (b, 0, 0)),
            pl.BlockSpec((_T, 1), lambda b: (0, 0)),
            pl.BlockSpec((_T, 1), lambda b: (0, 0)),
            pl.BlockSpec((_T, _C), lambda b: (0, 0)),
        ],
        out_specs=pl.BlockSpec((_T, _C), lambda b: (0, 0)),
        out_shape=jax.ShapeDtypeStruct((_T, _C), jnp.float32),
    )(od, p0_col, p1_col, x1)


def kernel(x, ln1_w, ln1_b, ln2_w, ln2_b, qkv_w, proj_w, gate_w, w1, b1,
           w2, b2):
    x2d = x.reshape(_T, _C)
    pos = jnp.arange(_T, dtype=jnp.float32)[:, None]
    inv_freq = 1.0 / (10000.0 ** (
        jnp.arange(0, _HD, 2, dtype=jnp.float32) / _HD))
    ang = pos * inv_freq
    sin = jnp.sin(ang)
    cos = jnp.cos(ang)

    ln1 = _ln_call(x2d, ln1_w, ln1_b)
    q, k, v = _qkv_call(ln1, qkv_w, sin, cos)
    ctx = _attn_call(q, k, v)
    x1, h2, inds2, vals2 = _proj_call(ctx, proj_w, x2d, ln2_w, ln2_b, gate_w)

    e32 = inds2.reshape(_SR, _SL)
    pos32, bexp2d = _route_call(e32)
    pos2 = pos32.reshape(_T, _TOPK)
    p0_col, p1_col = pos2[:, 0:1], pos2[:, 1:2]
    p0_row, p1_row = p0_col.reshape(1, _T), p1_col.reshape(1, _T)
    w0_row = vals2[:, 0].reshape(1, _T)
    w1_row = vals2[:, 1].reshape(1, _T)
    bexp = bexp2d.reshape(_SL)[:_NB]

    swt, xd = _dispatch_call(p0_row, p1_row, w0_row, w1_row,
                             h2.astype(jnp.bfloat16))
    od = _ffn_call(bexp, xd, w1, b1, w2, b2, swt)
    out = _combine_call(od, p0_col, p1_col, x1)
    return out.reshape(_B, _T, _C)
